# baseline (speedup 1.0000x reference)
import functools

import numpy as np
import jax
import jax.numpy as jnp
from jax import lax
from jax.experimental import pallas as pl
from jax.experimental.pallas import tpu as pltpu
import jax.experimental.pallas.tpu_sc as plsc

N_ENT = 10000
EMB = 256
NP = 10240
CHUNK = NP // 16
HALF = 80000
EPT = 5120
NREAL15 = EPT - (16 * EPT - HALF)

_ONES = np.ones((EPT,), np.float32)

_MESH = plsc.VectorSubcoreMesh(
    core_axis_name="c", subcore_axis_name="s", num_cores=2, num_subcores=16)


@functools.partial(
    pl.kernel,
    out_type=jax.ShapeDtypeStruct((8, NP), jnp.float32),
    mesh=_MESH,
    compiler_params=pltpu.CompilerParams(needs_layout_passes=False),
    scratch_types=[
        pltpu.VMEM((EPT,), jnp.int32),
        pltpu.VMEM((EPT,), jnp.int32),
        pltpu.VMEM((EPT,), jnp.float32),
        pltpu.VMEM((CHUNK,), jnp.float32),
        pltpu.VMEM((CHUNK,), jnp.float32),
        pltpu.VMEM_SHARED((NP,), jnp.float32),
        pltpu.VMEM_SHARED((NP,), jnp.float32),
        pltpu.SemaphoreType.DMA,
    ],
)
def _edge_scalars(row_hbm, col_hbm, ones_hbm, out_hbm,
                  row_v, col_v, val_v, cha, chb, sh_deg, sh_t, sem):
    c = lax.axis_index("c")
    s = lax.axis_index("s")

    dv = pltpu.async_copy(ones_hbm, val_v, sem)

    @pl.when(s < 15)
    def _():
        da = pltpu.async_copy(row_hbm.at[pl.ds(c * HALF + s * EPT, EPT)],
                              row_v, sem)
        db = pltpu.async_copy(col_hbm.at[pl.ds(c * HALF + s * EPT, EPT)],
                              col_v, sem)
        da.wait()
        db.wait()

    @pl.when(s == 15)
    def _():
        da = pltpu.async_copy(
            row_hbm.at[pl.ds(c * HALF + 15 * EPT, NREAL15)],
            row_v.at[pl.ds(0, NREAL15)], sem)
        db = pltpu.async_copy(
            col_hbm.at[pl.ds(c * HALF + 15 * EPT, NREAL15)],
            col_v.at[pl.ds(0, NREAL15)], sem)
        lanes = lax.iota(jnp.int32, 16)

        def _respread(k, carry):
            pad = N_ENT + ((k * 16 + lanes) % (NP - N_ENT))
            row_v[pl.ds(NREAL15 + k * 16, 16)] = pad
            col_v[pl.ds(NREAL15 + k * 16, 16)] = pad
            return carry
        lax.fori_loop(0, (EPT - NREAL15) // 16, _respread, 0)
        da.wait()
        db.wait()

    def _zero(i, carry):
        cha[pl.ds(i * 16, 16)] = jnp.zeros((16,), jnp.float32)
        return carry
    lax.fori_loop(0, CHUNK // 16, _zero, 0)
    dz1 = pltpu.async_copy(cha, sh_deg.at[pl.ds(s * CHUNK, CHUNK)], sem)
    dz2 = pltpu.async_copy(cha, sh_t.at[pl.ds(s * CHUNK, CHUNK)], sem)
    dz1.wait()
    dz2.wait()
    dv.wait()
    plsc.subcore_barrier()

    pltpu.sync_copy(val_v, sh_deg.at[row_v], add=True)
    plsc.subcore_barrier()

    pltpu.sync_copy(sh_deg.at[pl.ds(s * CHUNK, CHUNK)], cha)

    def _p2(i, carry):
        d = cha[pl.ds(i * 16, 16)]
        c1 = d >= 65536.0
        d1 = jnp.where(c1, d * (1.0 / 65536.0), d)
        r1 = jnp.where(c1, 1.0 / 256.0, 1.0)
        c2 = d1 >= 256.0
        d2 = jnp.where(c2, d1 * (1.0 / 256.0), d1)
        r2 = jnp.where(c2, 1.0 / 16.0, 1.0)
        c3 = d2 >= 16.0
        d3 = jnp.where(c3, d2 * (1.0 / 16.0), d2)
        r3 = jnp.where(c3, 0.25, 1.0)
        y = d3 * 0.25 + 0.97
        y = (y + d3 / y) * 0.5
        y = (y + d3 / y) * 0.5
        y = (y + d3 / y) * 0.5
        y = (y + d3 / y) * 0.5
        dinv = (r1 * r2 * r3) / y
        cha[pl.ds(i * 16, 16)] = jnp.where(d > 0.5, dinv, 0.0)
        return carry
    lax.fori_loop(0, CHUNK // 16, _p2, 0)
    pltpu.sync_copy(cha, sh_deg.at[pl.ds(s * CHUNK, CHUNK)])
    plsc.subcore_barrier()

    pltpu.sync_copy(sh_deg.at[col_v], val_v)
    pltpu.sync_copy(val_v, sh_t.at[row_v], add=True)
    plsc.subcore_barrier()

    da = pltpu.async_copy(sh_deg.at[pl.ds(s * CHUNK, CHUNK)], cha, sem)
    db = pltpu.async_copy(sh_t.at[pl.ds(s * CHUNK, CHUNK)], chb, sem)
    da.wait()
    db.wait()

    def _p4(i, carry):
        cha[pl.ds(i * 16, 16)] = cha[pl.ds(i * 16, 16)] * chb[pl.ds(i * 16, 16)]
        return carry
    lax.fori_loop(0, CHUNK // 16, _p4, 0)
    pltpu.sync_copy(cha, out_hbm.at[c, pl.ds(s * CHUNK, CHUNK)])


BM = 2000
NB = N_ENT // BM


def _fused_body(x_ref, win_ref, wout_ref, wloop_ref, s_ref,
                g_ref, b_ref, o_ref, pre_scr, scol_scr, acc_scr, stat_scr):
    p = pl.program_id(0)
    j = pl.program_id(1)

    @pl.when(p == 0)
    def _():
        @pl.when(j == 0)
        def _():
            scol_scr[...] = jnp.swapaxes(s_ref[...], 0, 1)
            acc_scr[...] = jnp.zeros_like(acc_scr)

        x = x_ref[...]
        scol = scol_scr[pl.ds(j * BM, BM), :]
        pre = (jnp.dot(x, win_ref[...]) * scol[:, 0:1]
               + jnp.dot(x, wout_ref[...]) * scol[:, 1:2]
               + jnp.dot(x, wloop_ref[...])) * (1.0 / 3.0)
        pre_scr[pl.ds(j * BM, BM), :] = pre

        acc_scr[0:1, :] += jnp.sum(pre, axis=0, keepdims=True)
        acc_scr[1:2, :] += jnp.sum(pre * pre, axis=0, keepdims=True)

        @pl.when(j == NB - 1)
        def _():
            mean = acc_scr[0:1, :] * (1.0 / N_ENT)
            var = acc_scr[1:2, :] * (1.0 / N_ENT) - mean * mean
            a = lax.rsqrt(var + 1e-5) * g_ref[...]
            stat_scr[0:1, :] = a
            stat_scr[1:2, :] = b_ref[...] - mean * a

    @pl.when(p == 1)
    def _():
        pre = pre_scr[pl.ds(j * BM, BM), :]
        o_ref[...] = jnp.tanh(pre * stat_scr[0:1, :] + stat_scr[1:2, :])


_fused = pl.pallas_call(
    _fused_body,
    grid=(2, NB),
    in_specs=[
        pl.BlockSpec((BM, EMB), lambda p, j: ((1 - p) * j, 0)),
        pl.BlockSpec((EMB, EMB), lambda p, j: (0, 0)),
        pl.BlockSpec((EMB, EMB), lambda p, j: (0, 0)),
        pl.BlockSpec((EMB, EMB), lambda p, j: (0, 0)),
        pl.BlockSpec((8, NP), lambda p, j: (0, 0)),
        pl.BlockSpec((1, EMB), lambda p, j: (0, 0)),
        pl.BlockSpec((1, EMB), lambda p, j: (0, 0)),
    ],
    out_specs=pl.BlockSpec((BM, EMB), lambda p, j: (p * j, 0)),
    out_shape=jax.ShapeDtypeStruct((N_ENT, EMB), jnp.float32),
    scratch_shapes=[
        pltpu.VMEM((N_ENT, EMB), jnp.float32),
        pltpu.VMEM((NP, 8), jnp.float32),
        pltpu.VMEM((2, EMB), jnp.float32),
        pltpu.VMEM((2, EMB), jnp.float32),
    ],
)


def kernel(x, rel_embed, edge_index, edge_type, w_in, w_out, w_loop,
           gamma, beta):
    s_all = _edge_scalars(edge_index[0], edge_index[1], jnp.asarray(_ONES))
    res = _fused(x, w_in, w_out, w_loop, s_all,
                 gamma.reshape(1, EMB), beta.reshape(1, EMB))
    return (res, rel_embed)


# speedup vs baseline: 1.0712x; 1.0712x over previous
import functools

import numpy as np
import jax
import jax.numpy as jnp
from jax import lax
from jax.experimental import pallas as pl
from jax.experimental.pallas import tpu as pltpu
import jax.experimental.pallas.tpu_sc as plsc

N_ENT = 10000
EMB = 256
NP = 10240
CHUNK = NP // 16
HALF = 80000
EPT = 5120
NREAL15 = EPT - (16 * EPT - HALF)

_ONES = np.ones((EPT,), np.float32)

_MESH = plsc.VectorSubcoreMesh(
    core_axis_name="c", subcore_axis_name="s", num_cores=2, num_subcores=16)


@functools.partial(
    pl.kernel,
    out_type=jax.ShapeDtypeStruct((8, NP), jnp.float32),
    mesh=_MESH,
    compiler_params=pltpu.CompilerParams(needs_layout_passes=False),
    scratch_types=[
        pltpu.VMEM((2, EPT), jnp.int32),
        pltpu.VMEM((EPT,), jnp.int32),
        pltpu.VMEM((EPT,), jnp.int32),
        pltpu.VMEM((EPT,), jnp.float32),
        pltpu.VMEM((CHUNK,), jnp.float32),
        pltpu.VMEM((CHUNK,), jnp.float32),
        pltpu.VMEM_SHARED((NP,), jnp.float32),
        pltpu.VMEM_SHARED((NP,), jnp.float32),
        pltpu.SemaphoreType.DMA,
    ],
)
def _edge_scalars(ei_hbm, ones_hbm, out_hbm,
                  ev, row_v, col_v, val_v, cha, chb, sh_deg, sh_t, sem):
    c = lax.axis_index("c")
    s = lax.axis_index("s")

    dv = pltpu.async_copy(ones_hbm, val_v, sem)

    @pl.when(s < 15)
    def _():
        da = pltpu.async_copy(
            ei_hbm.at[pl.ds(0, 2), pl.ds(c * HALF + s * EPT, EPT)], ev, sem)
        da.wait()

    @pl.when(s == 15)
    def _():
        da = pltpu.async_copy(
            ei_hbm.at[pl.ds(0, 2), pl.ds(c * HALF + 15 * EPT, NREAL15)],
            ev.at[pl.ds(0, 2), pl.ds(0, NREAL15)], sem)
        da.wait()

    def _zero(i, carry):
        cha[pl.ds(i * 16, 16)] = jnp.zeros((16,), jnp.float32)
        return carry
    lax.fori_loop(0, CHUNK // 16, _zero, 0)

    def _deint(i, carry):
        for u in range(8):
            off = pl.ds((i * 8 + u) * 16, 16)
            row_v[off] = ev[0, off]
            col_v[off] = ev[1, off]
        return carry
    lax.fori_loop(0, EPT // 128, _deint, 0)

    @pl.when(s == 15)
    def _():
        lanes = lax.iota(jnp.int32, 16)

        def _respread(k, carry):
            pad = N_ENT + ((k * 16 + lanes) % (NP - N_ENT))
            row_v[pl.ds(NREAL15 + k * 16, 16)] = pad
            col_v[pl.ds(NREAL15 + k * 16, 16)] = pad
            return carry
        lax.fori_loop(0, (EPT - NREAL15) // 16, _respread, 0)
    dz1 = pltpu.async_copy(cha, sh_deg.at[pl.ds(s * CHUNK, CHUNK)], sem)
    dz2 = pltpu.async_copy(cha, sh_t.at[pl.ds(s * CHUNK, CHUNK)], sem)
    dz1.wait()
    dz2.wait()
    dv.wait()
    plsc.subcore_barrier()

    pltpu.sync_copy(val_v, sh_deg.at[row_v], add=True)
    plsc.subcore_barrier()

    pltpu.sync_copy(sh_deg.at[pl.ds(s * CHUNK, CHUNK)], cha)

    def _p2(i, carry):
        d = cha[pl.ds(i * 16, 16)]
        c1 = d >= 65536.0
        d1 = jnp.where(c1, d * (1.0 / 65536.0), d)
        r1 = jnp.where(c1, 1.0 / 256.0, 1.0)
        c2 = d1 >= 256.0
        d2 = jnp.where(c2, d1 * (1.0 / 256.0), d1)
        r2 = jnp.where(c2, 1.0 / 16.0, 1.0)
        c3 = d2 >= 16.0
        d3 = jnp.where(c3, d2 * (1.0 / 16.0), d2)
        r3 = jnp.where(c3, 0.25, 1.0)
        y = d3 * 0.25 + 0.97
        y = (y + d3 / y) * 0.5
        y = (y + d3 / y) * 0.5
        y = (y + d3 / y) * 0.5
        y = (y + d3 / y) * 0.5
        dinv = (r1 * r2 * r3) / y
        cha[pl.ds(i * 16, 16)] = jnp.where(d > 0.5, dinv, 0.0)
        return carry
    lax.fori_loop(0, CHUNK // 16, _p2, 0)
    pltpu.sync_copy(cha, sh_deg.at[pl.ds(s * CHUNK, CHUNK)])
    plsc.subcore_barrier()

    pltpu.sync_copy(sh_deg.at[col_v], val_v)
    pltpu.sync_copy(val_v, sh_t.at[row_v], add=True)
    plsc.subcore_barrier()

    da = pltpu.async_copy(sh_deg.at[pl.ds(s * CHUNK, CHUNK)], cha, sem)
    db = pltpu.async_copy(sh_t.at[pl.ds(s * CHUNK, CHUNK)], chb, sem)
    da.wait()
    db.wait()

    def _p4(i, carry):
        cha[pl.ds(i * 16, 16)] = cha[pl.ds(i * 16, 16)] * chb[pl.ds(i * 16, 16)]
        return carry
    lax.fori_loop(0, CHUNK // 16, _p4, 0)
    pltpu.sync_copy(cha, out_hbm.at[c, pl.ds(s * CHUNK, CHUNK)])


BM = 2000
NB = N_ENT // BM


def _fused_body(x_ref, win_ref, wout_ref, wloop_ref, s_ref,
                g_ref, b_ref, o_ref, pre_scr, scol_scr, acc_scr, stat_scr):
    p = pl.program_id(0)
    j = pl.program_id(1)

    @pl.when(p == 0)
    def _():
        @pl.when(j == 0)
        def _():
            scol_scr[...] = jnp.swapaxes(s_ref[...], 0, 1)
            acc_scr[...] = jnp.zeros_like(acc_scr)

        x = x_ref[...]
        scol = scol_scr[pl.ds(j * BM, BM), :]
        pre = (jnp.dot(x, win_ref[...]) * scol[:, 0:1]
               + jnp.dot(x, wout_ref[...]) * scol[:, 1:2]
               + jnp.dot(x, wloop_ref[...])) * (1.0 / 3.0)
        pre_scr[pl.ds(j * BM, BM), :] = pre

        acc_scr[0:1, :] += jnp.sum(pre, axis=0, keepdims=True)
        acc_scr[1:2, :] += jnp.sum(pre * pre, axis=0, keepdims=True)

        @pl.when(j == NB - 1)
        def _():
            mean = acc_scr[0:1, :] * (1.0 / N_ENT)
            var = acc_scr[1:2, :] * (1.0 / N_ENT) - mean * mean
            a = lax.rsqrt(var + 1e-5) * g_ref[...]
            stat_scr[0:1, :] = a
            stat_scr[1:2, :] = b_ref[...] - mean * a

    @pl.when(p == 1)
    def _():
        pre = pre_scr[pl.ds(j * BM, BM), :]
        o_ref[...] = jnp.tanh(pre * stat_scr[0:1, :] + stat_scr[1:2, :])


_fused = pl.pallas_call(
    _fused_body,
    grid=(2, NB),
    in_specs=[
        pl.BlockSpec((BM, EMB), lambda p, j: ((1 - p) * j, 0)),
        pl.BlockSpec((EMB, EMB), lambda p, j: (0, 0)),
        pl.BlockSpec((EMB, EMB), lambda p, j: (0, 0)),
        pl.BlockSpec((EMB, EMB), lambda p, j: (0, 0)),
        pl.BlockSpec((8, NP), lambda p, j: (0, 0)),
        pl.BlockSpec((1, EMB), lambda p, j: (0, 0)),
        pl.BlockSpec((1, EMB), lambda p, j: (0, 0)),
    ],
    out_specs=pl.BlockSpec((BM, EMB), lambda p, j: (p * j, 0)),
    out_shape=jax.ShapeDtypeStruct((N_ENT, EMB), jnp.float32),
    scratch_shapes=[
        pltpu.VMEM((N_ENT, EMB), jnp.float32),
        pltpu.VMEM((NP, 8), jnp.float32),
        pltpu.VMEM((2, EMB), jnp.float32),
        pltpu.VMEM((2, EMB), jnp.float32),
    ],
)


def kernel(x, rel_embed, edge_index, edge_type, w_in, w_out, w_loop,
           gamma, beta):
    s_all = _edge_scalars(edge_index, jnp.asarray(_ONES))
    res = _fused(x, w_in, w_out, w_loop, s_all,
                 gamma.reshape(1, EMB), beta.reshape(1, EMB))
    return (res, rel_embed)


# speedup vs baseline: 1.1036x; 1.0302x over previous
import functools

import numpy as np
import jax
import jax.numpy as jnp
from jax import lax
from jax.experimental import pallas as pl
from jax.experimental.pallas import tpu as pltpu
import jax.experimental.pallas.tpu_sc as plsc

N_ENT = 10000
EMB = 256
NP = 10240
CHUNK = NP // 16
HALF = 80000
EPT = 5120
NREAL15 = EPT - (16 * EPT - HALF)

_ONES = np.ones((EPT,), np.float32)

_MESH = plsc.VectorSubcoreMesh(
    core_axis_name="c", subcore_axis_name="s", num_cores=2, num_subcores=16)


@functools.partial(
    pl.kernel,
    out_type=jax.ShapeDtypeStruct((8, NP), jnp.float32),
    mesh=_MESH,
    compiler_params=pltpu.CompilerParams(needs_layout_passes=False),
    scratch_types=[
        pltpu.VMEM((EPT,), jnp.int32),
        pltpu.VMEM((EPT,), jnp.int32),
        pltpu.VMEM((EPT,), jnp.float32),
        pltpu.VMEM((CHUNK,), jnp.float32),
        pltpu.VMEM((CHUNK,), jnp.float32),
        pltpu.VMEM_SHARED((NP,), jnp.float32),
        pltpu.VMEM_SHARED((NP,), jnp.float32),
        pltpu.SemaphoreType.DMA,
    ],
)
def _edge_scalars(ei_hbm, ones_hbm, out_hbm,
                  row_v, col_v, val_v, cha, chb, sh_deg, sh_t, sem):
    c = lax.axis_index("c")
    s = lax.axis_index("s")

    dv = pltpu.async_copy(ones_hbm, val_v, sem)

    @pl.when(s < 15)
    def _():
        da = pltpu.async_copy(
            ei_hbm.at[pl.ds(c * HALF + s * EPT, EPT)], row_v, sem)
        db = pltpu.async_copy(
            ei_hbm.at[pl.ds(2 * HALF + c * HALF + s * EPT, EPT)], col_v, sem)
        da.wait()
        db.wait()

    @pl.when(s == 15)
    def _():
        da = pltpu.async_copy(
            ei_hbm.at[pl.ds(c * HALF + 15 * EPT, NREAL15)],
            row_v.at[pl.ds(0, NREAL15)], sem)
        db = pltpu.async_copy(
            ei_hbm.at[pl.ds(2 * HALF + c * HALF + 15 * EPT, NREAL15)],
            col_v.at[pl.ds(0, NREAL15)], sem)
        da.wait()
        db.wait()

    def _zero(i, carry):
        cha[pl.ds(i * 16, 16)] = jnp.zeros((16,), jnp.float32)
        return carry
    lax.fori_loop(0, CHUNK // 16, _zero, 0)


    @pl.when(s == 15)
    def _():
        lanes = lax.iota(jnp.int32, 16)

        def _respread(k, carry):
            pad = N_ENT + ((k * 16 + lanes) % (NP - N_ENT))
            row_v[pl.ds(NREAL15 + k * 16, 16)] = pad
            col_v[pl.ds(NREAL15 + k * 16, 16)] = pad
            return carry
        lax.fori_loop(0, (EPT - NREAL15) // 16, _respread, 0)
    dz1 = pltpu.async_copy(cha, sh_deg.at[pl.ds(s * CHUNK, CHUNK)], sem)
    dz2 = pltpu.async_copy(cha, sh_t.at[pl.ds(s * CHUNK, CHUNK)], sem)
    dz1.wait()
    dz2.wait()
    dv.wait()
    plsc.subcore_barrier()

    pltpu.sync_copy(val_v, sh_deg.at[row_v], add=True)
    plsc.subcore_barrier()

    pltpu.sync_copy(sh_deg.at[pl.ds(s * CHUNK, CHUNK)], cha)

    def _p2(i, carry):
        d = cha[pl.ds(i * 16, 16)]
        c1 = d >= 65536.0
        d1 = jnp.where(c1, d * (1.0 / 65536.0), d)
        r1 = jnp.where(c1, 1.0 / 256.0, 1.0)
        c2 = d1 >= 256.0
        d2 = jnp.where(c2, d1 * (1.0 / 256.0), d1)
        r2 = jnp.where(c2, 1.0 / 16.0, 1.0)
        c3 = d2 >= 16.0
        d3 = jnp.where(c3, d2 * (1.0 / 16.0), d2)
        r3 = jnp.where(c3, 0.25, 1.0)
        y = d3 * 0.25 + 0.97
        y = (y + d3 / y) * 0.5
        y = (y + d3 / y) * 0.5
        y = (y + d3 / y) * 0.5
        y = (y + d3 / y) * 0.5
        dinv = (r1 * r2 * r3) / y
        cha[pl.ds(i * 16, 16)] = jnp.where(d > 0.5, dinv, 0.0)
        return carry
    lax.fori_loop(0, CHUNK // 16, _p2, 0)
    pltpu.sync_copy(cha, sh_deg.at[pl.ds(s * CHUNK, CHUNK)])
    plsc.subcore_barrier()

    pltpu.sync_copy(sh_deg.at[col_v], val_v)
    pltpu.sync_copy(val_v, sh_t.at[row_v], add=True)
    plsc.subcore_barrier()

    da = pltpu.async_copy(sh_deg.at[pl.ds(s * CHUNK, CHUNK)], cha, sem)
    db = pltpu.async_copy(sh_t.at[pl.ds(s * CHUNK, CHUNK)], chb, sem)
    da.wait()
    db.wait()

    def _p4(i, carry):
        cha[pl.ds(i * 16, 16)] = cha[pl.ds(i * 16, 16)] * chb[pl.ds(i * 16, 16)]
        return carry
    lax.fori_loop(0, CHUNK // 16, _p4, 0)
    pltpu.sync_copy(cha, out_hbm.at[c, pl.ds(s * CHUNK, CHUNK)])


BM = 2000
NB = N_ENT // BM


def _fused_body(x_ref, win_ref, wout_ref, wloop_ref, s_ref,
                g_ref, b_ref, o_ref, pre_scr, scol_scr, acc_scr, stat_scr):
    p = pl.program_id(0)
    j = pl.program_id(1)

    @pl.when(p == 0)
    def _():
        @pl.when(j == 0)
        def _():
            scol_scr[...] = jnp.swapaxes(s_ref[...], 0, 1)
            acc_scr[...] = jnp.zeros_like(acc_scr)

        x = x_ref[...]
        scol = scol_scr[pl.ds(j * BM, BM), :]
        pre = (jnp.dot(x, win_ref[...]) * scol[:, 0:1]
               + jnp.dot(x, wout_ref[...]) * scol[:, 1:2]
               + jnp.dot(x, wloop_ref[...])) * (1.0 / 3.0)
        pre_scr[pl.ds(j * BM, BM), :] = pre

        acc_scr[0:1, :] += jnp.sum(pre, axis=0, keepdims=True)
        acc_scr[1:2, :] += jnp.sum(pre * pre, axis=0, keepdims=True)

        @pl.when(j == NB - 1)
        def _():
            mean = acc_scr[0:1, :] * (1.0 / N_ENT)
            var = acc_scr[1:2, :] * (1.0 / N_ENT) - mean * mean
            a = lax.rsqrt(var + 1e-5) * g_ref[...]
            stat_scr[0:1, :] = a
            stat_scr[1:2, :] = b_ref[...] - mean * a

    @pl.when(p == 1)
    def _():
        pre = pre_scr[pl.ds(j * BM, BM), :]
        o_ref[...] = jnp.tanh(pre * stat_scr[0:1, :] + stat_scr[1:2, :])


_fused = pl.pallas_call(
    _fused_body,
    grid=(2, NB),
    in_specs=[
        pl.BlockSpec((BM, EMB), lambda p, j: ((1 - p) * j, 0)),
        pl.BlockSpec((EMB, EMB), lambda p, j: (0, 0)),
        pl.BlockSpec((EMB, EMB), lambda p, j: (0, 0)),
        pl.BlockSpec((EMB, EMB), lambda p, j: (0, 0)),
        pl.BlockSpec((8, NP), lambda p, j: (0, 0)),
        pl.BlockSpec((1, EMB), lambda p, j: (0, 0)),
        pl.BlockSpec((1, EMB), lambda p, j: (0, 0)),
    ],
    out_specs=pl.BlockSpec((BM, EMB), lambda p, j: (p * j, 0)),
    out_shape=jax.ShapeDtypeStruct((N_ENT, EMB), jnp.float32),
    scratch_shapes=[
        pltpu.VMEM((N_ENT, EMB), jnp.float32),
        pltpu.VMEM((NP, 8), jnp.float32),
        pltpu.VMEM((2, EMB), jnp.float32),
        pltpu.VMEM((2, EMB), jnp.float32),
    ],
)


def kernel(x, rel_embed, edge_index, edge_type, w_in, w_out, w_loop,
           gamma, beta):
    s_all = _edge_scalars(edge_index.reshape(4 * HALF), jnp.asarray(_ONES))
    res = _fused(x, w_in, w_out, w_loop, s_all,
                 gamma.reshape(1, EMB), beta.reshape(1, EMB))
    return (res, rel_embed)
